# baseline (speedup 1.0000x reference)
import functools

import jax
import jax.numpy as jnp
from jax import lax
from jax.experimental import pallas as pl
from jax.experimental.pallas import tpu as pltpu
from jax.experimental.pallas import tpu_sc as plsc

B = 16384
D = 10
N = 1000000
L = 16
NC = 2
NS = 16
NW = NC * NS
BPW = B // NW
NB = 26


def _rsqrt(x):
    i = lax.bitcast_convert_type(x, jnp.int32)
    i = jnp.int32(0x5F3759DF) - lax.shift_right_arithmetic(i, jnp.int32(1))
    y = lax.bitcast_convert_type(i, jnp.float32)
    for _ in range(3):
        y = y * (jnp.float32(1.5) - jnp.float32(0.5) * x * y * y)
    return y


_mesh = plsc.VectorSubcoreMesh(core_axis_name="c", subcore_axis_name="s")


@functools.partial(
    pl.kernel,
    mesh=_mesh,
    out_type=jax.ShapeDtypeStruct((B,), jnp.float32),
    compiler_params=pltpu.CompilerParams(needs_layout_passes=False),
    scratch_types=[
        pltpu.VMEM((BPW,), jnp.int32),
        pltpu.VMEM((BPW,), jnp.int32),
        pltpu.VMEM((BPW,), jnp.int32),
        pltpu.VMEM((BPW,), jnp.int32),
        pltpu.VMEM((NB, D, 128), jnp.float32),
        pltpu.VMEM((NB, D, 128), jnp.float32),
        pltpu.VMEM((D, BPW), jnp.float32),
        pltpu.VMEM((D, BPW), jnp.float32),
        pltpu.VMEM((BPW,), jnp.float32),
        pltpu.SemaphoreType.DMA,
        pltpu.SemaphoreType.DMA,
    ],
)
def _cosine_kernel(o_id_hbm, e_id_hbm, mentors_t_hbm, mentees_t_hbm, out_hbm,
                   oidx_v, eidx_v, omod_v, emod_v, oring_v, ering_v,
                   ocols_v, ecols_v, res_v, semo, seme):
    wid = lax.axis_index("s") * NC + lax.axis_index("c")
    base = wid * BPW

    pltpu.sync_copy(o_id_hbm.at[pl.ds(base, BPW)], oidx_v)
    pltpu.sync_copy(e_id_hbm.at[pl.ds(base, BPW)], eidx_v)

    lanes = lax.iota(jnp.int32, L)
    dmask = lanes < D

    def mod_body(g, _):
        sl = pl.ds(g * L, L)
        omod_v[sl] = oidx_v[sl] & 127
        emod_v[sl] = eidx_v[sl] & 127
        return 0

    lax.fori_loop(0, BPW // L, mod_body, 0)

    def _scalar_at(vref, j):
        gb = pl.multiple_of(lax.shift_left(lax.shift_right_logical(j, 4), 4),
                            L)
        vec = vref[pl.ds(gb, L)]
        lane = j & 15
        return jnp.sum(jnp.where(lanes == lane, vec, 0))

    def fire(j, b):
        ro = _scalar_at(oidx_v, j)
        re = _scalar_at(eidx_v, j)
        co = pl.multiple_of(lax.shift_left(lax.shift_right_logical(ro, 7), 7),
                            128)
        ce = pl.multiple_of(lax.shift_left(lax.shift_right_logical(re, 7), 7),
                            128)
        pltpu.async_copy(mentors_t_hbm.at[:, pl.ds(co, 128)], oring_v.at[b],
                         semo)
        pltpu.async_copy(mentees_t_hbm.at[:, pl.ds(ce, 128)], ering_v.at[b],
                         seme)

    def drain_and_extract(j, b):
        pltpu.make_async_copy(mentors_t_hbm.at[:, pl.ds(0, 128)],
                              oring_v.at[b], semo).wait()
        pltpu.make_async_copy(mentees_t_hbm.at[:, pl.ds(0, 128)],
                              ering_v.at[b], seme).wait()
        gb = pl.multiple_of(lax.shift_left(lax.shift_right_logical(j, 4), 4),
                            L)
        lane = jnp.full((L,), j & 15, jnp.int32)
        mo = jnp.take(omod_v[pl.ds(gb, L)], lane)
        me = jnp.take(emod_v[pl.ds(gb, L)], lane)
        jv = jnp.full((L,), j, jnp.int32)
        ov = plsc.load_gather(oring_v.at[b], [lanes, mo], mask=dmask)
        ev = plsc.load_gather(ering_v.at[b], [lanes, me], mask=dmask)
        plsc.store_scatter(ocols_v, [lanes, jv], ov, mask=dmask)
        plsc.store_scatter(ecols_v, [lanes, jv], ev, mask=dmask)

    for j in range(NB):
        fire(j, j)

    def main_body(j, _):
        b = lax.rem(j, NB)
        drain_and_extract(j, b)
        fire(j + NB, b)
        return 0

    lax.fori_loop(0, BPW - NB, main_body, 0)

    def tail_body(j, _):
        drain_and_extract(j, lax.rem(j, NB))
        return 0

    lax.fori_loop(BPW - NB, BPW, tail_body, 0)

    zero = jnp.zeros((L,), jnp.float32)

    def cos_body(g, _):
        rbase = pl.multiple_of(g * L, L)
        sl = pl.ds(rbase, L)
        dot = zero
        on2 = zero
        en2 = zero
        for d in range(D):
            o = ocols_v[d, sl]
            e = ecols_v[d, sl]
            dot = dot + o * e
            on2 = on2 + o * o
            en2 = en2 + e * e
        res_v[sl] = dot * _rsqrt(on2 * en2)
        return 0

    lax.fori_loop(0, BPW // L, cos_body, 0)

    pltpu.sync_copy(res_v, out_hbm.at[pl.ds(base, BPW)])


def kernel(o_id, e_id, mentors, mentees):
    return _cosine_kernel(o_id.astype(jnp.int32), e_id.astype(jnp.int32),
                          mentors.T, mentees.T)


# speedup vs baseline: 1.0064x; 1.0064x over previous
import functools

import jax
import jax.numpy as jnp
from jax import lax
from jax.experimental import pallas as pl
from jax.experimental.pallas import tpu as pltpu
from jax.experimental.pallas import tpu_sc as plsc

B = 16384
D = 10
N = 1000000
L = 16
NC = 2
NS = 16
NW = NC * NS
BPW = B // NW
NB = 24


def _rsqrt(x):
    i = lax.bitcast_convert_type(x, jnp.int32)
    i = jnp.int32(0x5F3759DF) - lax.shift_right_arithmetic(i, jnp.int32(1))
    y = lax.bitcast_convert_type(i, jnp.float32)
    for _ in range(3):
        y = y * (jnp.float32(1.5) - jnp.float32(0.5) * x * y * y)
    return y


_mesh = plsc.VectorSubcoreMesh(core_axis_name="c", subcore_axis_name="s")


@functools.partial(
    pl.kernel,
    mesh=_mesh,
    out_type=jax.ShapeDtypeStruct((B,), jnp.float32),
    compiler_params=pltpu.CompilerParams(needs_layout_passes=False),
    scratch_types=[
        pltpu.VMEM((BPW,), jnp.int32),
        pltpu.VMEM((BPW,), jnp.int32),
        pltpu.VMEM((BPW,), jnp.int32),
        pltpu.VMEM((BPW,), jnp.int32),
        pltpu.VMEM((NB, D, 128), jnp.float32),
        pltpu.VMEM((NB, D, 128), jnp.float32),
        pltpu.VMEM((D, BPW), jnp.float32),
        pltpu.VMEM((D, BPW), jnp.float32),
        pltpu.VMEM((BPW,), jnp.float32),
        pltpu.SemaphoreType.DMA,
        pltpu.SemaphoreType.DMA,
    ],
)
def _cosine_kernel(o_id_hbm, e_id_hbm, mentors_t_hbm, mentees_t_hbm, out_hbm,
                   oidx_v, eidx_v, omod_v, emod_v, oring_v, ering_v,
                   ocols_v, ecols_v, res_v, semo, seme):
    wid = lax.axis_index("s") * NC + lax.axis_index("c")
    base = wid * BPW

    pltpu.sync_copy(o_id_hbm.at[pl.ds(base, BPW)], oidx_v)
    pltpu.sync_copy(e_id_hbm.at[pl.ds(base, BPW)], eidx_v)

    lanes = lax.iota(jnp.int32, L)
    dmask = lanes < D

    def mod_body(g, _):
        sl = pl.ds(g * L, L)
        omod_v[sl] = oidx_v[sl] & 127
        emod_v[sl] = eidx_v[sl] & 127
        return 0

    lax.fori_loop(0, BPW // L, mod_body, 0)

    def _scalar_at(vref, j):
        gb = pl.multiple_of(lax.shift_left(lax.shift_right_logical(j, 4), 4),
                            L)
        vec = vref[pl.ds(gb, L)]
        lane = j & 15
        return jnp.sum(jnp.where(lanes == lane, vec, 0))

    def fire(j, b):
        ro = _scalar_at(oidx_v, j)
        re = _scalar_at(eidx_v, j)
        co = pl.multiple_of(lax.shift_left(lax.shift_right_logical(ro, 7), 7),
                            128)
        ce = pl.multiple_of(lax.shift_left(lax.shift_right_logical(re, 7), 7),
                            128)
        pltpu.async_copy(mentors_t_hbm.at[:, pl.ds(co, 128)], oring_v.at[b],
                         semo)
        pltpu.async_copy(mentees_t_hbm.at[:, pl.ds(ce, 128)], ering_v.at[b],
                         seme)

    def drain_and_extract(j, b):
        pltpu.make_async_copy(mentors_t_hbm.at[:, pl.ds(0, 128)],
                              oring_v.at[b], semo).wait()
        pltpu.make_async_copy(mentees_t_hbm.at[:, pl.ds(0, 128)],
                              ering_v.at[b], seme).wait()
        gb = pl.multiple_of(lax.shift_left(lax.shift_right_logical(j, 4), 4),
                            L)
        lane = jnp.full((L,), j & 15, jnp.int32)
        mo = jnp.take(omod_v[pl.ds(gb, L)], lane)
        me = jnp.take(emod_v[pl.ds(gb, L)], lane)
        jv = jnp.full((L,), j, jnp.int32)
        ov = plsc.load_gather(oring_v.at[b], [lanes, mo], mask=dmask)
        ev = plsc.load_gather(ering_v.at[b], [lanes, me], mask=dmask)
        plsc.store_scatter(ocols_v, [lanes, jv], ov, mask=dmask)
        plsc.store_scatter(ecols_v, [lanes, jv], ev, mask=dmask)

    for j in range(NB):
        fire(j, j)

    def main_body(j, _):
        b = lax.rem(j, NB)
        drain_and_extract(j, b)
        fire(j + NB, b)
        return 0

    lax.fori_loop(0, BPW - NB, main_body, 0)

    def tail_body(j, _):
        drain_and_extract(j, lax.rem(j, NB))
        return 0

    lax.fori_loop(BPW - NB, BPW, tail_body, 0)

    zero = jnp.zeros((L,), jnp.float32)

    def cos_body(g, _):
        rbase = pl.multiple_of(g * L, L)
        sl = pl.ds(rbase, L)
        dot = zero
        on2 = zero
        en2 = zero
        for d in range(D):
            o = ocols_v[d, sl]
            e = ecols_v[d, sl]
            dot = dot + o * e
            on2 = on2 + o * o
            en2 = en2 + e * e
        res_v[sl] = dot * _rsqrt(on2 * en2)
        return 0

    lax.fori_loop(0, BPW // L, cos_body, 0)

    pltpu.sync_copy(res_v, out_hbm.at[pl.ds(base, BPW)])


def kernel(o_id, e_id, mentors, mentees):
    return _cosine_kernel(o_id.astype(jnp.int32), e_id.astype(jnp.int32),
                          mentors.T, mentees.T)
